# baseline (speedup 1.0000x reference)
import functools

import jax
import jax.numpy as jnp
from jax import lax
from jax.experimental import pallas as pl
from jax.experimental.pallas import tpu as pltpu
from jax.experimental.pallas import tpu_sc as plsc

D = 64
LANES = 16
CB = 8


def _make_encoder(batch, seq, n_deg):
    info = plsc.get_sparse_core_info()
    nc, ns = info.num_cores, info.num_subcores
    nw = nc * ns
    b_per_w = batch // nw
    assert batch % nw == 0 and b_per_w % CB == 0
    n_chunks = b_per_w // CB

    mesh = plsc.VectorSubcoreMesh(core_axis_name="c", subcore_axis_name="s")

    @functools.partial(
        pl.kernel,
        mesh=mesh,
        compiler_params=pltpu.CompilerParams(
            use_tc_tiling_on_sc=False, needs_layout_passes=False),
        out_type=jax.ShapeDtypeStruct((batch, seq, D), jnp.float32),
        scratch_types=[
            pltpu.VMEM((b_per_w, seq), jnp.int32),
            pltpu.VMEM((b_per_w * seq,), jnp.int32),
            pltpu.VMEM((n_deg, D), jnp.float32),
            pltpu.VMEM((CB, seq, D), jnp.float32),
            pltpu.VMEM((CB, seq, D), jnp.float32),
            pltpu.SemaphoreType.DMA,
            pltpu.SemaphoreType.DMA,
            pltpu.SemaphoreType.DMA,
        ],
    )
    def enc(node_hbm, deg_hbm, nidx_hbm, didx_hbm, out_hbm,
            nidx_v, didx_v, deg_v, rows_a, rows_b, nsem, osem_a, osem_b):
        wid = lax.axis_index("s") * nc + lax.axis_index("c")
        base = wid * b_per_w
        bufs = (rows_a, rows_b)
        osems = (osem_a, osem_b)

        pltpu.sync_copy(deg_hbm, deg_v)
        pltpu.sync_copy(nidx_hbm.at[pl.ds(base, b_per_w)], nidx_v)
        pltpu.sync_copy(didx_hbm.at[pl.ds(base * seq, b_per_w * seq)], didx_v)

        def fire_node(ci):
            buf = bufs[ci % 2]
            return [
                pltpu.async_copy(
                    node_hbm.at[nidx_v.at[ci * CB + j]], buf.at[j], nsem)
                for j in range(CB)
            ]

        def out_copy(ci):
            return pltpu.make_async_copy(
                bufs[ci % 2], out_hbm.at[pl.ds(base + ci * CB, CB)],
                osems[ci % 2])

        node_cps = fire_node(0)
        for ci in range(n_chunks):
            for cp in node_cps:
                cp.wait()
            node_cps = []
            if ci + 1 < n_chunks:
                if ci >= 1:
                    out_copy(ci - 1).wait()
                node_cps = fire_node(ci + 1)

            buf = bufs[ci % 2]

            def s_body(s, c2, ci=ci, buf=buf):
                iota = lax.iota(jnp.int32, LANES)
                for j in range(CB):
                    fr = (ci * CB + j) * seq + s
                    di16 = plsc.load_gather(
                        didx_v, [jnp.full((LANES,), fr, jnp.int32)])
                    for c in range(D // LANES):
                        sl = pl.ds(c * LANES, LANES)
                        dvals = plsc.load_gather(
                            deg_v, [di16, iota + c * LANES])
                        buf[j, s, sl] = buf[j, s, sl] + dvals
                return c2

            lax.fori_loop(0, seq, s_body, 0)
            out_copy(ci).start()
        out_copy(n_chunks - 2).wait()
        out_copy(n_chunks - 1).wait()

    return enc


@jax.jit
def kernel(user_seq, user_degree, node_table, degree_table):
    b, s = user_seq.shape
    enc = _make_encoder(b, s, degree_table.shape[0])
    return enc(node_table, degree_table, user_seq, user_degree.reshape(-1))


# speedup vs baseline: 1.2294x; 1.2294x over previous
import functools

import jax
import jax.numpy as jnp
from jax import lax
from jax.experimental import pallas as pl
from jax.experimental.pallas import tpu as pltpu
from jax.experimental.pallas import tpu_sc as plsc

D = 64
CB = 16


def _make_encoder(batch, seq):
    info = plsc.get_sparse_core_info()
    nc, ns = info.num_cores, info.num_subcores
    nw = nc * ns
    b_per_w = batch // nw
    assert batch % nw == 0 and b_per_w % CB == 0
    n_chunks = b_per_w // CB
    chunk_128rows = CB * seq * D // 128

    mesh = plsc.VectorSubcoreMesh(core_axis_name="c", subcore_axis_name="s")

    @functools.partial(
        pl.kernel,
        mesh=mesh,
        compiler_params=pltpu.CompilerParams(use_tc_tiling_on_sc=False),
        out_type=jax.ShapeDtypeStruct((batch, seq, D), jnp.float32),
        scratch_types=[
            pltpu.VMEM((CB, seq), jnp.int32),
            pltpu.VMEM((CB, seq), jnp.int32),
            pltpu.VMEM((CB, seq, D), jnp.float32),
            pltpu.SemaphoreType.DMA,
            pltpu.SemaphoreType.DMA,
        ],
    )
    def enc(node_hbm, deg_hbm, nidx_hbm, didx_hbm, out_hbm,
            nidx_v, didx_v, rows_v, nsem, dsem):
        wid = lax.axis_index("s") * nc + lax.axis_index("c")
        base = wid * b_per_w

        def chunk_body(ci, carry):
            b0 = base + ci * CB
            pltpu.sync_copy(nidx_hbm.at[pl.ds(b0, CB)], nidx_v)
            pltpu.sync_copy(didx_hbm.at[pl.ds(b0, CB)], didx_v)
            copies = []
            for j in range(CB):
                copies.append(pltpu.async_copy(
                    node_hbm.at[nidx_v.at[j]], rows_v.at[j], nsem))
            for cp in copies:
                cp.wait()
            copies = []
            for j in range(CB):
                copies.append(pltpu.async_copy(
                    deg_hbm.at[didx_v.at[j]], rows_v.at[j], dsem, add=True))
            for cp in copies:
                cp.wait()
            pltpu.sync_copy(rows_v, out_hbm.at[pl.ds(b0, CB)])
            return carry

        lax.fori_loop(0, n_chunks, chunk_body, 0)

    return enc


@jax.jit
def kernel(user_seq, user_degree, node_table, degree_table):
    b, s = user_seq.shape
    enc = _make_encoder(b, s)
    return enc(node_table, degree_table, user_seq, user_degree)
